# baseline (speedup 1.0000x reference)
import functools

import jax
import jax.numpy as jnp
from jax import lax
from jax.experimental import pallas as pl
from jax.experimental.pallas import tpu as pltpu
from jax.experimental.pallas import tpu_sc as plsc

N_NODES = 10000
D_FEAT = 128
N_EDGES = 320000

NUM_CORES = 2
NUM_SUBCORES = 16
COLS = D_FEAT // NUM_CORES
CNT_W = 16
COLS_G = COLS + CNT_W

CHUNK = 64
CHUNKS_PER_TILE = 314
E_PAD = NUM_SUBCORES * CHUNKS_PER_TILE * CHUNK
N_ACC = 10240
ROWS_PER_TILE = N_ACC // NUM_SUBCORES
ZSTEPS = ROWS_PER_TILE // CHUNK
LANES = 16


def _sc_aggregate(eb, xs):
  mesh = plsc.VectorSubcoreMesh(core_axis_name="c", subcore_axis_name="s")

  @functools.partial(
      pl.kernel,
      out_type=jax.ShapeDtypeStruct((NUM_CORES, N_ACC, COLS_G), jnp.float32),
      mesh=mesh,
      compiler_params=pltpu.CompilerParams(use_tc_tiling_on_sc=False),
      scratch_types=[
          pltpu.VMEM((2, 2, CHUNK), jnp.int32),
          pltpu.VMEM((CHUNK, COLS_G), jnp.float32),
          pltpu.VMEM((CHUNK, COLS_G), jnp.float32),
          pltpu.VMEM_SHARED((N_ACC, COLS_G), jnp.float32),
          pltpu.SemaphoreType.DMA,
          pltpu.SemaphoreType.DMA,
          pltpu.SemaphoreType.DMA,
          pltpu.SemaphoreType.DMA,
      ],
  )
  def agg(eb_hbm, xs_hbm,
          sums_out,
          sd2, rows0, rows1,
          ssum, gsem0, gsem1, ssem0, ssem1):
    c = lax.axis_index("c")
    s = lax.axis_index("s")

    zero16 = jnp.zeros((LANES,), jnp.float32)

    @pl.loop(0, CHUNK)
    def _(i):
      @pl.loop(0, COLS_G // LANES)
      def _(j):
        rows0[i, pl.ds(j * LANES, LANES)] = zero16

    @pl.loop(0, ZSTEPS)
    def _(k):
      row0 = s * ROWS_PER_TILE + k * CHUNK
      pltpu.sync_copy(rows0, ssum.at[pl.ds(row0, CHUNK)])

    plsc.subcore_barrier()

    cbase = s * CHUNKS_PER_TILE

    def load_idx(slot, i):
      pltpu.sync_copy(eb_hbm.at[cbase + i], sd2.at[slot])

    def start_gather(slot, rows, sem):
      pltpu.async_copy(xs_hbm.at[c].at[sd2.at[slot].at[0]], rows, sem)

    def wait_gather(slot, rows, sem):
      pltpu.make_async_copy(
          xs_hbm.at[c].at[sd2.at[slot].at[0]], rows, sem).wait()

    def start_scatter(slot, rows, sem):
      pltpu.async_copy(rows, ssum.at[sd2.at[slot].at[1]], sem, add=True)

    def wait_scatter(slot, rows, sem):
      pltpu.make_async_copy(rows, ssum.at[sd2.at[slot].at[1]], sem).wait()

    load_idx(0, 0)
    start_gather(0, rows0, gsem0)

    @pl.loop(0, CHUNKS_PER_TILE, step=2)
    def _(i):
      @pl.when(i > 0)
      def _():
        wait_scatter(1, rows1, ssem1)

      load_idx(1, i + 1)
      start_gather(1, rows1, gsem1)
      wait_gather(0, rows0, gsem0)
      start_scatter(0, rows0, ssem0)

      @pl.when(i + 2 < CHUNKS_PER_TILE)
      def _():
        wait_scatter(0, rows0, ssem0)
        load_idx(0, i + 2)
        start_gather(0, rows0, gsem0)

      wait_gather(1, rows1, gsem1)
      start_scatter(1, rows1, ssem1)

    wait_scatter(0, rows0, ssem0)
    wait_scatter(1, rows1, ssem1)

    plsc.subcore_barrier()

    @pl.loop(0, ZSTEPS)
    def _(k):
      row0 = s * ROWS_PER_TILE + k * CHUNK
      pltpu.sync_copy(ssum.at[pl.ds(row0, CHUNK)], rows0)
      pltpu.sync_copy(rows0, sums_out.at[c, pl.ds(row0, CHUNK)])

  return agg(eb, xs)


def _finish(sums, W, b):
  blk = 1000

  def body(s_ref, w_ref, b_ref, o_ref):
    sm = jnp.concatenate(
        [s_ref[0, :, :COLS], s_ref[1, :, :COLS]], axis=1)
    mean = sm / jnp.maximum(s_ref[0, :, COLS:COLS + 1], 1.0)
    o_ref[...] = (
        jnp.dot(mean, w_ref[...], preferred_element_type=jnp.float32)
        + b_ref[...]
    )

  return pl.pallas_call(
      body,
      grid=(N_NODES // blk,),
      in_specs=[
          pl.BlockSpec((NUM_CORES, blk, COLS_G), lambda i: (0, i, 0)),
          pl.BlockSpec((D_FEAT, D_FEAT), lambda i: (0, 0)),
          pl.BlockSpec((1, D_FEAT), lambda i: (0, 0)),
      ],
      out_specs=pl.BlockSpec((blk, D_FEAT), lambda i: (i, 0)),
      out_shape=jax.ShapeDtypeStruct((N_NODES, D_FEAT), jnp.float32),
  )(sums, W, b.reshape(1, D_FEAT))


def kernel(x, edge_index, W, b):
  ei = edge_index.astype(jnp.int32)
  pad = E_PAD - N_EDGES
  src = jnp.concatenate([ei[0], jnp.zeros((pad,), jnp.int32)])
  dst = jnp.concatenate([ei[1], jnp.full((pad,), N_NODES, jnp.int32)])
  eb = jnp.stack(
      [src.reshape(-1, CHUNK), dst.reshape(-1, CHUNK)], axis=1)
  ones = jnp.ones((N_NODES, CNT_W), jnp.float32)
  xs = jnp.stack([
      jnp.concatenate([x[:, :COLS], ones], axis=1),
      jnp.concatenate([x[:, COLS:], ones], axis=1),
  ])
  sums = _sc_aggregate(eb, xs)
  return _finish(sums, W, b)


# speedup vs baseline: 1.1519x; 1.1519x over previous
import functools

import jax
import jax.numpy as jnp
from jax import lax
from jax.experimental import pallas as pl
from jax.experimental.pallas import tpu as pltpu
from jax.experimental.pallas import tpu_sc as plsc

N_NODES = 10000
D_FEAT = 128
N_EDGES = 320000

NUM_CORES = 2
NUM_SUBCORES = 16
COLS = D_FEAT // NUM_CORES

CHUNK = 128
CHUNKS_PER_TILE = 158
E_PAD = NUM_SUBCORES * CHUNKS_PER_TILE * CHUNK
N_ACC = 10240
ROWS_PER_TILE = N_ACC // NUM_SUBCORES
ZSTEPS = ROWS_PER_TILE // CHUNK
CNT_W = 16
LANES = 16


def _sc_aggregate(eb, xs):
  mesh = plsc.VectorSubcoreMesh(core_axis_name="c", subcore_axis_name="s")

  @functools.partial(
      pl.kernel,
      out_type=(
          jax.ShapeDtypeStruct((NUM_CORES, N_ACC, COLS), jnp.float32),
          jax.ShapeDtypeStruct((NUM_CORES, N_ACC, CNT_W), jnp.float32),
      ),
      mesh=mesh,
      compiler_params=pltpu.CompilerParams(use_tc_tiling_on_sc=False),
      scratch_types=[
          pltpu.VMEM((2, 2, CHUNK), jnp.int32),
          pltpu.VMEM((CHUNK, COLS), jnp.float32),
          pltpu.VMEM((CHUNK, COLS), jnp.float32),
          pltpu.VMEM((CHUNK, CNT_W), jnp.float32),
          pltpu.VMEM((CHUNK, CNT_W), jnp.float32),
          pltpu.VMEM_SHARED((N_ACC, COLS), jnp.float32),
          pltpu.VMEM_SHARED((N_ACC, CNT_W), jnp.float32),
          pltpu.SemaphoreType.DMA,
          pltpu.SemaphoreType.DMA,
          pltpu.SemaphoreType.DMA,
          pltpu.SemaphoreType.DMA,
      ],
  )
  def agg(eb_hbm, xs_hbm,
          sums_out, cnts_out,
          sd2, rows0, rows1, ones_v, cbuf,
          ssum, scnt, gsem0, gsem1, ssem0, ssem1):
    c = lax.axis_index("c")
    s = lax.axis_index("s")

    zero16 = jnp.zeros((LANES,), jnp.float32)
    one16 = jnp.ones((LANES,), jnp.float32)

    @pl.loop(0, CHUNK)
    def _(i):
      @pl.loop(0, COLS // LANES)
      def _(j):
        rows0[i, pl.ds(j * LANES, LANES)] = zero16
      cbuf[i, :] = zero16
      ones_v[i, :] = one16

    @pl.loop(0, ZSTEPS)
    def _(k):
      row0 = s * ROWS_PER_TILE + k * CHUNK
      pltpu.sync_copy(rows0, ssum.at[pl.ds(row0, CHUNK)])
      pltpu.sync_copy(cbuf, scnt.at[pl.ds(row0, CHUNK)])

    plsc.subcore_barrier()

    cbase = s * CHUNKS_PER_TILE

    def load_idx(slot, i):
      pltpu.sync_copy(eb_hbm.at[cbase + i], sd2.at[slot])

    def start_gather(slot, rows, sem):
      pltpu.async_copy(xs_hbm.at[c].at[sd2.at[slot].at[0]], rows, sem)

    def wait_gather(slot, rows, sem):
      pltpu.make_async_copy(
          xs_hbm.at[c].at[sd2.at[slot].at[0]], rows, sem).wait()

    def start_scatter(slot, rows, sem):
      pltpu.async_copy(rows, ssum.at[sd2.at[slot].at[1]], sem, add=True)

      @pl.when(c == 0)
      def _():
        pltpu.async_copy(ones_v, scnt.at[sd2.at[slot].at[1]], sem, add=True)

    def wait_scatter(slot, rows, sem):
      pltpu.make_async_copy(rows, ssum.at[sd2.at[slot].at[1]], sem).wait()

      @pl.when(c == 0)
      def _():
        pltpu.make_async_copy(ones_v, scnt.at[sd2.at[slot].at[1]], sem).wait()

    load_idx(0, 0)
    start_gather(0, rows0, gsem0)

    @pl.loop(0, CHUNKS_PER_TILE, step=2)
    def _(i):
      @pl.when(i > 0)
      def _():
        wait_scatter(1, rows1, ssem1)

      load_idx(1, i + 1)
      start_gather(1, rows1, gsem1)
      wait_gather(0, rows0, gsem0)
      start_scatter(0, rows0, ssem0)

      @pl.when(i + 2 < CHUNKS_PER_TILE)
      def _():
        wait_scatter(0, rows0, ssem0)
        load_idx(0, i + 2)
        start_gather(0, rows0, gsem0)

      wait_gather(1, rows1, gsem1)
      start_scatter(1, rows1, ssem1)

    wait_scatter(0, rows0, ssem0)
    wait_scatter(1, rows1, ssem1)

    plsc.subcore_barrier()

    @pl.loop(0, ZSTEPS)
    def _(k):
      row0 = s * ROWS_PER_TILE + k * CHUNK
      pltpu.sync_copy(ssum.at[pl.ds(row0, CHUNK)], rows0)
      pltpu.sync_copy(rows0, sums_out.at[c, pl.ds(row0, CHUNK)])
      pltpu.sync_copy(scnt.at[pl.ds(row0, CHUNK)], cbuf)
      pltpu.sync_copy(cbuf, cnts_out.at[c, pl.ds(row0, CHUNK)])

  return agg(eb, xs)


def _finish(sums, cnts, W, b):
  blk = 1000

  def body(s_ref, c_ref, w_ref, b_ref, o_ref):
    sm = jnp.concatenate([s_ref[0], s_ref[1]], axis=1)
    mean = sm / jnp.maximum(c_ref[0][:, 0:1], 1.0)
    o_ref[...] = (
        jnp.dot(mean, w_ref[...], preferred_element_type=jnp.float32)
        + b_ref[...]
    )

  return pl.pallas_call(
      body,
      grid=(N_NODES // blk,),
      in_specs=[
          pl.BlockSpec((NUM_CORES, blk, COLS), lambda i: (0, i, 0)),
          pl.BlockSpec((1, blk, CNT_W), lambda i: (0, i, 0)),
          pl.BlockSpec((D_FEAT, D_FEAT), lambda i: (0, 0)),
          pl.BlockSpec((1, D_FEAT), lambda i: (0, 0)),
      ],
      out_specs=pl.BlockSpec((blk, D_FEAT), lambda i: (i, 0)),
      out_shape=jax.ShapeDtypeStruct((N_NODES, D_FEAT), jnp.float32),
  )(sums, cnts, W, b.reshape(1, D_FEAT))


def kernel(x, edge_index, W, b):
  ei = edge_index.astype(jnp.int32)
  pad = E_PAD - N_EDGES
  src = jnp.concatenate([ei[0], jnp.zeros((pad,), jnp.int32)])
  dst = jnp.concatenate([ei[1], jnp.full((pad,), N_NODES, jnp.int32)])
  eb = jnp.stack(
      [src.reshape(-1, CHUNK), dst.reshape(-1, CHUNK)], axis=1)
  xs = jnp.stack([x[:, :COLS], x[:, COLS:]])
  sums, cnts = _sc_aggregate(eb, xs)
  return _finish(sums, cnts, W, b)
